# Initial kernel scaffold; baseline (speedup 1.0000x reference)
import jax
import jax.numpy as jnp
from jax.experimental import pallas as pl


def kernel(coordinate, feature, W0, b0, g0, be0, rm0, rv0, W1, b1, g1, be1, rm1, rv1, W2, b2, g2, be2, rm2, rv2, Wf, bf):
    raise NotImplementedError("write your pallas kernel here")


# speedup vs baseline: 10.3311x; 10.3311x over previous
import functools

import jax
import jax.numpy as jnp
from jax import lax
from jax.experimental import pallas as pl
from jax.experimental.pallas import tpu as pltpu
from jax.experimental.pallas import tpu_sc as plsc

B, M, D = 4, 8192, 64
S = 512
K = 32
DP = 128
S_CHUNK = 128

NW = 32
ROWS_PER_W = (B * S * K) // NW
CHUNK = 512


def _mlp_topk_kernel(x_ref, w0_ref, b0_ref, g0_ref, be0_ref, rm0_ref, rv0_ref,
                     w1_ref, b1_ref, g1_ref, be1_ref, rm1_ref, rv1_ref,
                     w2_ref, b2_ref, g2_ref, be2_ref, rm2_ref, rv2_ref,
                     wf_ref, bf_ref, idx_ref):
    b = pl.program_id(0)
    h = x_ref[...]
    layers = ((w0_ref, b0_ref, g0_ref, be0_ref, rm0_ref, rv0_ref),
              (w1_ref, b1_ref, g1_ref, be1_ref, rm1_ref, rv1_ref),
              (w2_ref, b2_ref, g2_ref, be2_ref, rm2_ref, rv2_ref))
    for (w, bb, g, be, rm, rv) in layers:
        y = lax.dot_general(w[...], h, (((1,), (0,)), ((), ())),
                            preferred_element_type=jnp.float32)
        y = y + bb[...]
        y = (y - rm[...]) / jnp.sqrt(rv[...] + 1e-5) * g[...] + be[...]
        h = jnp.maximum(y, 0.0)
    iota = lax.broadcasted_iota(jnp.int32, (S_CHUNK, M), 1)
    for c in range(S // S_CHUNK):
        s0 = c * S_CHUNK
        q = lax.dot_general(wf_ref[s0:s0 + S_CHUNK, :], h,
                            (((1,), (0,)), ((), ())),
                            preferred_element_type=jnp.float32)
        q = q + bf_ref[s0:s0 + S_CHUNK, :]
        q = jax.nn.sigmoid(q)
        cols = []
        for _ in range(K):
            mx = jnp.max(q, axis=1, keepdims=True)
            cand = jnp.where(q == mx, iota, M)
            sel = jnp.min(cand, axis=1, keepdims=True)
            cols.append(sel)
            q = jnp.where(iota == sel, -1.0, q)
        idx_ref[s0:s0 + S_CHUNK, :] = jnp.concatenate(cols, axis=1) + b * M


def _run_mlp_topk(x_pad, w0p, b0, g0, be0, rm0, rv0, w1, b1, g1, be1, rm1, rv1,
                  w2, b2, g2, be2, rm2, rv2, wf, bf):
    full = lambda shape: pl.BlockSpec(shape, lambda b: (0,) * len(shape))
    specs = [pl.BlockSpec((None, 8, M), lambda b: (b, 0, 0))]
    for arr in (w0p, b0, g0, be0, rm0, rv0, w1, b1, g1, be1, rm1, rv1,
                w2, b2, g2, be2, rm2, rv2, wf, bf):
        specs.append(full(arr.shape))
    return pl.pallas_call(
        _mlp_topk_kernel,
        grid=(B,),
        in_specs=specs,
        out_specs=pl.BlockSpec((None, S, K), lambda b: (b, 0, 0)),
        out_shape=jax.ShapeDtypeStruct((B, S, K), jnp.int32),
    )(x_pad, w0p, b0, g0, be0, rm0, rv0, w1, b1, g1, be1, rm1, rv1,
      w2, b2, g2, be2, rm2, rv2, wf, bf)


@functools.cache
def _make_gather():
    mesh = plsc.VectorSubcoreMesh(core_axis_name="c", subcore_axis_name="s")

    @functools.partial(
        pl.kernel, mesh=mesh,
        out_type=jax.ShapeDtypeStruct((B * S * K, DP), jnp.float32),
        scratch_types=[
            pltpu.VMEM((CHUNK,), jnp.int32),
            pltpu.VMEM((CHUNK, DP), jnp.float32),
            pltpu.SemaphoreType.DMA,
        ],
    )
    def gather(table_hbm, idx_hbm, out_hbm, idx_v, rows_v, sem):
        wid = lax.axis_index("s") * 2 + lax.axis_index("c")
        base = wid * ROWS_PER_W
        for cstep in range(ROWS_PER_W // CHUNK):
            off = base + cstep * CHUNK
            pltpu.sync_copy(idx_hbm.at[pl.ds(off, CHUNK)], idx_v)
            pltpu.async_copy(table_hbm.at[idx_v], rows_v, sem).wait()
            pltpu.sync_copy(rows_v, out_hbm.at[pl.ds(off, CHUNK)])

    return gather


def kernel(coordinate, feature, W0, b0, g0, be0, rm0, rv0,
           W1, b1, g1, be1, rm1, rv1, W2, b2, g2, be2, rm2, rv2, Wf, bf):
    xT = jnp.transpose(coordinate, (0, 2, 1))
    x_pad = jnp.pad(xT, ((0, 0), (0, 5), (0, 0)))
    w0p = jnp.pad(W0, ((0, 0), (0, 5)))
    col = lambda p: p.reshape(-1, 1)
    idx = _run_mlp_topk(
        x_pad, w0p, col(b0), col(g0), col(be0), col(rm0), col(rv0),
        W1, col(b1), col(g1), col(be1), col(rm1), col(rv1),
        W2, col(b2), col(g2), col(be2), col(rm2), col(rv2),
        Wf, col(bf))
    table = jnp.pad(jnp.concatenate([coordinate, feature], axis=-1),
                    ((0, 0), (0, 0), (0, DP - 3 - D))).reshape(B * M, DP)
    rows = _make_gather()(table, idx.reshape(-1)).reshape(B, S, K, DP)
    grouped_points = rows[..., 0:3]
    grouped_feature = rows[..., 3:3 + D]
    return (grouped_points[:, :, 0, :], grouped_points,
            grouped_feature[:, :, 0, :], grouped_feature)


# speedup vs baseline: 10.3336x; 1.0002x over previous
import functools

import jax
import jax.numpy as jnp
from jax import lax
from jax.experimental import pallas as pl
from jax.experimental.pallas import tpu as pltpu
from jax.experimental.pallas import tpu_sc as plsc

B, M, D = 4, 8192, 64
S = 512
K = 32
DP = 128
S_CHUNK = 128

NW = 32
ROWS_PER_W = (B * S * K) // NW
CHUNK = 256


def _mlp_topk_kernel(x_ref, w0_ref, b0_ref, g0_ref, be0_ref, rm0_ref, rv0_ref,
                     w1_ref, b1_ref, g1_ref, be1_ref, rm1_ref, rv1_ref,
                     w2_ref, b2_ref, g2_ref, be2_ref, rm2_ref, rv2_ref,
                     wf_ref, bf_ref, idx_ref):
    b = pl.program_id(0)
    h = x_ref[...]
    layers = ((w0_ref, b0_ref, g0_ref, be0_ref, rm0_ref, rv0_ref),
              (w1_ref, b1_ref, g1_ref, be1_ref, rm1_ref, rv1_ref),
              (w2_ref, b2_ref, g2_ref, be2_ref, rm2_ref, rv2_ref))
    for (w, bb, g, be, rm, rv) in layers:
        y = lax.dot_general(w[...], h, (((1,), (0,)), ((), ())),
                            preferred_element_type=jnp.float32)
        y = y + bb[...]
        y = (y - rm[...]) / jnp.sqrt(rv[...] + 1e-5) * g[...] + be[...]
        h = jnp.maximum(y, 0.0)
    iota = lax.broadcasted_iota(jnp.int32, (S_CHUNK, M), 1)
    for c in range(S // S_CHUNK):
        s0 = c * S_CHUNK
        q = lax.dot_general(wf_ref[s0:s0 + S_CHUNK, :], h,
                            (((1,), (0,)), ((), ())),
                            preferred_element_type=jnp.float32)
        q = q + bf_ref[s0:s0 + S_CHUNK, :]
        q = jax.nn.sigmoid(q)
        cols = []
        for _ in range(K):
            mx = jnp.max(q, axis=1, keepdims=True)
            cand = jnp.where(q == mx, iota, M)
            sel = jnp.min(cand, axis=1, keepdims=True)
            cols.append(sel)
            q = jnp.where(iota == sel, -1.0, q)
        idx_ref[s0:s0 + S_CHUNK, :] = jnp.concatenate(cols, axis=1) + b * M


def _run_mlp_topk(x_pad, w0p, b0, g0, be0, rm0, rv0, w1, b1, g1, be1, rm1, rv1,
                  w2, b2, g2, be2, rm2, rv2, wf, bf):
    full = lambda shape: pl.BlockSpec(shape, lambda b: (0,) * len(shape))
    specs = [pl.BlockSpec((None, 8, M), lambda b: (b, 0, 0))]
    for arr in (w0p, b0, g0, be0, rm0, rv0, w1, b1, g1, be1, rm1, rv1,
                w2, b2, g2, be2, rm2, rv2, wf, bf):
        specs.append(full(arr.shape))
    return pl.pallas_call(
        _mlp_topk_kernel,
        grid=(B,),
        in_specs=specs,
        out_specs=pl.BlockSpec((None, S, K), lambda b: (b, 0, 0)),
        out_shape=jax.ShapeDtypeStruct((B, S, K), jnp.int32),
    )(x_pad, w0p, b0, g0, be0, rm0, rv0, w1, b1, g1, be1, rm1, rv1,
      w2, b2, g2, be2, rm2, rv2, wf, bf)


@functools.cache
def _make_gather():
    mesh = plsc.VectorSubcoreMesh(core_axis_name="c", subcore_axis_name="s")

    @functools.partial(
        pl.kernel, mesh=mesh,
        out_type=jax.ShapeDtypeStruct((B * S * K, DP), jnp.float32),
        scratch_types=[
            pltpu.VMEM((CHUNK,), jnp.int32),
            pltpu.VMEM((CHUNK,), jnp.int32),
            pltpu.VMEM((CHUNK, DP), jnp.float32),
            pltpu.VMEM((CHUNK, DP), jnp.float32),
            pltpu.SemaphoreType.DMA,
            pltpu.SemaphoreType.DMA,
        ],
    )
    def gather(table_hbm, idx_hbm, out_hbm,
               idx_v0, idx_v1, rows_v0, rows_v1, sem0, sem1):
        wid = lax.axis_index("s") * 2 + lax.axis_index("c")
        base = wid * ROWS_PER_W
        nstep = ROWS_PER_W // CHUNK
        idx_b = (idx_v0, idx_v1)
        rows_b = (rows_v0, rows_v1)
        sem_b = (sem0, sem1)
        pltpu.sync_copy(idx_hbm.at[pl.ds(base, CHUNK)], idx_v0)
        cps = [pltpu.async_copy(table_hbm.at[idx_v0], rows_v0, sem0)]
        for cstep in range(nstep):
            if cstep + 1 < nstep:
                noff = base + (cstep + 1) * CHUNK
                nb = (cstep + 1) % 2
                pltpu.sync_copy(idx_hbm.at[pl.ds(noff, CHUNK)], idx_b[nb])
                cps.append(pltpu.async_copy(
                    table_hbm.at[idx_b[nb]], rows_b[nb], sem_b[nb]))
            cps[cstep].wait()
            pltpu.sync_copy(rows_b[cstep % 2],
                            out_hbm.at[pl.ds(base + cstep * CHUNK, CHUNK)])

    return gather


def kernel(coordinate, feature, W0, b0, g0, be0, rm0, rv0,
           W1, b1, g1, be1, rm1, rv1, W2, b2, g2, be2, rm2, rv2, Wf, bf):
    xT = jnp.transpose(coordinate, (0, 2, 1))
    x_pad = jnp.pad(xT, ((0, 0), (0, 5), (0, 0)))
    w0p = jnp.pad(W0, ((0, 0), (0, 5)))
    col = lambda p: p.reshape(-1, 1)
    idx = _run_mlp_topk(
        x_pad, w0p, col(b0), col(g0), col(be0), col(rm0), col(rv0),
        W1, col(b1), col(g1), col(be1), col(rm1), col(rv1),
        W2, col(b2), col(g2), col(be2), col(rm2), col(rv2),
        Wf, col(bf))
    table = jnp.pad(jnp.concatenate([coordinate, feature], axis=-1),
                    ((0, 0), (0, 0), (0, DP - 3 - D))).reshape(B * M, DP)
    rows = _make_gather()(table, idx.reshape(-1)).reshape(B, S, K, DP)
    grouped_points = rows[..., 0:3]
    grouped_feature = rows[..., 3:3 + D]
    return (grouped_points[:, :, 0, :], grouped_points,
            grouped_feature[:, :, 0, :], grouped_feature)


# speedup vs baseline: 10.3421x; 1.0008x over previous
import functools

import jax
import jax.numpy as jnp
from jax import lax
from jax.experimental import pallas as pl
from jax.experimental.pallas import tpu as pltpu
from jax.experimental.pallas import tpu_sc as plsc

B, M, D = 4, 8192, 64
S = 512
K = 32
DP = 128
S_CHUNK = 256

NW = 32
ROWS_PER_W = (B * S * K) // NW
CHUNK = 256


def _mlp_topk_kernel(x_ref, w0_ref, b0_ref, g0_ref, be0_ref, rm0_ref, rv0_ref,
                     w1_ref, b1_ref, g1_ref, be1_ref, rm1_ref, rv1_ref,
                     w2_ref, b2_ref, g2_ref, be2_ref, rm2_ref, rv2_ref,
                     wf_ref, bf_ref, idx_ref):
    b = pl.program_id(0)
    h = x_ref[...]
    layers = ((w0_ref, b0_ref, g0_ref, be0_ref, rm0_ref, rv0_ref),
              (w1_ref, b1_ref, g1_ref, be1_ref, rm1_ref, rv1_ref),
              (w2_ref, b2_ref, g2_ref, be2_ref, rm2_ref, rv2_ref))
    for (w, bb, g, be, rm, rv) in layers:
        y = lax.dot_general(w[...], h, (((1,), (0,)), ((), ())),
                            preferred_element_type=jnp.float32)
        y = y + bb[...]
        y = (y - rm[...]) / jnp.sqrt(rv[...] + 1e-5) * g[...] + be[...]
        h = jnp.maximum(y, 0.0)
    iota = lax.broadcasted_iota(jnp.int32, (S_CHUNK, M), 1)
    for c in range(S // S_CHUNK):
        s0 = c * S_CHUNK
        q = lax.dot_general(wf_ref[s0:s0 + S_CHUNK, :], h,
                            (((1,), (0,)), ((), ())),
                            preferred_element_type=jnp.float32)
        q = q + bf_ref[s0:s0 + S_CHUNK, :]
        q = jax.nn.sigmoid(q)
        cols = []
        for _ in range(K):
            mx = jnp.max(q, axis=1, keepdims=True)
            cand = jnp.where(q == mx, iota, M)
            sel = jnp.min(cand, axis=1, keepdims=True)
            cols.append(sel)
            q = jnp.where(iota == sel, -1.0, q)
        idx_ref[s0:s0 + S_CHUNK, :] = jnp.concatenate(cols, axis=1) + b * M


def _run_mlp_topk(x_pad, w0p, b0, g0, be0, rm0, rv0, w1, b1, g1, be1, rm1, rv1,
                  w2, b2, g2, be2, rm2, rv2, wf, bf):
    full = lambda shape: pl.BlockSpec(shape, lambda b: (0,) * len(shape))
    specs = [pl.BlockSpec((None, 8, M), lambda b: (b, 0, 0))]
    for arr in (w0p, b0, g0, be0, rm0, rv0, w1, b1, g1, be1, rm1, rv1,
                w2, b2, g2, be2, rm2, rv2, wf, bf):
        specs.append(full(arr.shape))
    return pl.pallas_call(
        _mlp_topk_kernel,
        grid=(B,),
        in_specs=specs,
        out_specs=pl.BlockSpec((None, S, K), lambda b: (b, 0, 0)),
        out_shape=jax.ShapeDtypeStruct((B, S, K), jnp.int32),
    )(x_pad, w0p, b0, g0, be0, rm0, rv0, w1, b1, g1, be1, rm1, rv1,
      w2, b2, g2, be2, rm2, rv2, wf, bf)


@functools.cache
def _make_gather():
    mesh = plsc.VectorSubcoreMesh(core_axis_name="c", subcore_axis_name="s")

    @functools.partial(
        pl.kernel, mesh=mesh,
        out_type=jax.ShapeDtypeStruct((B * S * K, DP), jnp.float32),
        scratch_types=[
            pltpu.VMEM((CHUNK,), jnp.int32),
            pltpu.VMEM((CHUNK,), jnp.int32),
            pltpu.VMEM((CHUNK, DP), jnp.float32),
            pltpu.VMEM((CHUNK, DP), jnp.float32),
            pltpu.SemaphoreType.DMA,
            pltpu.SemaphoreType.DMA,
        ],
    )
    def gather(table_hbm, idx_hbm, out_hbm,
               idx_v0, idx_v1, rows_v0, rows_v1, sem0, sem1):
        wid = lax.axis_index("s") * 2 + lax.axis_index("c")
        base = wid * ROWS_PER_W
        nstep = ROWS_PER_W // CHUNK
        idx_b = (idx_v0, idx_v1)
        rows_b = (rows_v0, rows_v1)
        sem_b = (sem0, sem1)
        pltpu.sync_copy(idx_hbm.at[pl.ds(base, CHUNK)], idx_v0)
        cps = [pltpu.async_copy(table_hbm.at[idx_v0], rows_v0, sem0)]
        for cstep in range(nstep):
            if cstep + 1 < nstep:
                noff = base + (cstep + 1) * CHUNK
                nb = (cstep + 1) % 2
                pltpu.sync_copy(idx_hbm.at[pl.ds(noff, CHUNK)], idx_b[nb])
                cps.append(pltpu.async_copy(
                    table_hbm.at[idx_b[nb]], rows_b[nb], sem_b[nb]))
            cps[cstep].wait()
            pltpu.sync_copy(rows_b[cstep % 2],
                            out_hbm.at[pl.ds(base + cstep * CHUNK, CHUNK)])

    return gather


def kernel(coordinate, feature, W0, b0, g0, be0, rm0, rv0,
           W1, b1, g1, be1, rm1, rv1, W2, b2, g2, be2, rm2, rv2, Wf, bf):
    xT = jnp.transpose(coordinate, (0, 2, 1))
    x_pad = jnp.pad(xT, ((0, 0), (0, 5), (0, 0)))
    w0p = jnp.pad(W0, ((0, 0), (0, 5)))
    col = lambda p: p.reshape(-1, 1)
    idx = _run_mlp_topk(
        x_pad, w0p, col(b0), col(g0), col(be0), col(rm0), col(rv0),
        W1, col(b1), col(g1), col(be1), col(rm1), col(rv1),
        W2, col(b2), col(g2), col(be2), col(rm2), col(rv2),
        Wf, col(bf))
    table = jnp.pad(jnp.concatenate([coordinate, feature], axis=-1),
                    ((0, 0), (0, 0), (0, DP - 3 - D))).reshape(B * M, DP)
    rows = _make_gather()(table, idx.reshape(-1)).reshape(B, S, K, DP)
    grouped_points = rows[..., 0:3]
    grouped_feature = rows[..., 3:3 + D]
    return (grouped_points[:, :, 0, :], grouped_points,
            grouped_feature[:, :, 0, :], grouped_feature)


# speedup vs baseline: 10.3684x; 1.0025x over previous
import functools

import jax
import jax.numpy as jnp
from jax import lax
from jax.experimental import pallas as pl
from jax.experimental.pallas import tpu as pltpu
from jax.experimental.pallas import tpu_sc as plsc

B, M, D = 4, 8192, 64
S = 512
K = 32
DP = 128
S_CHUNK = 256

NW = 32
ROWS_PER_W = (B * S * K) // NW
CHUNK = 256


def _mlp_topk_kernel(x_ref, w0_ref, b0_ref, g0_ref, be0_ref, rm0_ref, rv0_ref,
                     w1_ref, b1_ref, g1_ref, be1_ref, rm1_ref, rv1_ref,
                     w2_ref, b2_ref, g2_ref, be2_ref, rm2_ref, rv2_ref,
                     wf_ref, bf_ref, idx_ref):
    b = pl.program_id(0)
    h = x_ref[...]
    layers = ((w0_ref, b0_ref, g0_ref, be0_ref, rm0_ref, rv0_ref),
              (w1_ref, b1_ref, g1_ref, be1_ref, rm1_ref, rv1_ref),
              (w2_ref, b2_ref, g2_ref, be2_ref, rm2_ref, rv2_ref))
    for (w, bb, g, be, rm, rv) in layers:
        y = lax.dot_general(w[...], h, (((1,), (0,)), ((), ())),
                            preferred_element_type=jnp.float32)
        y = y + bb[...]
        y = (y - rm[...]) / jnp.sqrt(rv[...] + 1e-5) * g[...] + be[...]
        h = jnp.maximum(y, 0.0)
    iota = lax.broadcasted_iota(jnp.int32, (S_CHUNK, M), 1)
    nchunks = S // S_CHUNK

    def _score(c):
        s0 = c * S_CHUNK
        q = lax.dot_general(wf_ref[s0:s0 + S_CHUNK, :], h,
                            (((1,), (0,)), ((), ())),
                            preferred_element_type=jnp.float32)
        return jax.nn.sigmoid(q + bf_ref[s0:s0 + S_CHUNK, :])

    q = _score(0)
    for c in range(nchunks):
        q_next = _score(c + 1) if c + 1 < nchunks else None
        cols = []
        for _ in range(K):
            mx = jnp.max(q, axis=1, keepdims=True)
            cand = jnp.where(q == mx, iota, M)
            sel = jnp.min(cand, axis=1, keepdims=True)
            cols.append(sel)
            q = jnp.where(iota == sel, -1.0, q)
        s0 = c * S_CHUNK
        idx_ref[s0:s0 + S_CHUNK, :] = jnp.concatenate(cols, axis=1) + b * M
        q = q_next


def _run_mlp_topk(x_pad, w0p, b0, g0, be0, rm0, rv0, w1, b1, g1, be1, rm1, rv1,
                  w2, b2, g2, be2, rm2, rv2, wf, bf):
    full = lambda shape: pl.BlockSpec(shape, lambda b: (0,) * len(shape))
    specs = [pl.BlockSpec((None, 8, M), lambda b: (b, 0, 0))]
    for arr in (w0p, b0, g0, be0, rm0, rv0, w1, b1, g1, be1, rm1, rv1,
                w2, b2, g2, be2, rm2, rv2, wf, bf):
        specs.append(full(arr.shape))
    return pl.pallas_call(
        _mlp_topk_kernel,
        grid=(B,),
        in_specs=specs,
        out_specs=pl.BlockSpec((None, S, K), lambda b: (b, 0, 0)),
        out_shape=jax.ShapeDtypeStruct((B, S, K), jnp.int32),
    )(x_pad, w0p, b0, g0, be0, rm0, rv0, w1, b1, g1, be1, rm1, rv1,
      w2, b2, g2, be2, rm2, rv2, wf, bf)


@functools.cache
def _make_gather():
    mesh = plsc.VectorSubcoreMesh(core_axis_name="c", subcore_axis_name="s")

    @functools.partial(
        pl.kernel, mesh=mesh,
        out_type=jax.ShapeDtypeStruct((B * S * K, DP), jnp.float32),
        scratch_types=[
            pltpu.VMEM((CHUNK,), jnp.int32),
            pltpu.VMEM((CHUNK,), jnp.int32),
            pltpu.VMEM((CHUNK, DP), jnp.float32),
            pltpu.VMEM((CHUNK, DP), jnp.float32),
            pltpu.SemaphoreType.DMA,
            pltpu.SemaphoreType.DMA,
        ],
    )
    def gather(table_hbm, idx_hbm, out_hbm,
               idx_v0, idx_v1, rows_v0, rows_v1, sem0, sem1):
        wid = lax.axis_index("s") * 2 + lax.axis_index("c")
        base = wid * ROWS_PER_W
        nstep = ROWS_PER_W // CHUNK
        idx_b = (idx_v0, idx_v1)
        rows_b = (rows_v0, rows_v1)
        sem_b = (sem0, sem1)
        pltpu.sync_copy(idx_hbm.at[pl.ds(base, CHUNK)], idx_v0)
        cps = [pltpu.async_copy(table_hbm.at[idx_v0], rows_v0, sem0)]
        for cstep in range(nstep):
            if cstep + 1 < nstep:
                noff = base + (cstep + 1) * CHUNK
                nb = (cstep + 1) % 2
                pltpu.sync_copy(idx_hbm.at[pl.ds(noff, CHUNK)], idx_b[nb])
                cps.append(pltpu.async_copy(
                    table_hbm.at[idx_b[nb]], rows_b[nb], sem_b[nb]))
            cps[cstep].wait()
            pltpu.sync_copy(rows_b[cstep % 2],
                            out_hbm.at[pl.ds(base + cstep * CHUNK, CHUNK)])

    return gather


def kernel(coordinate, feature, W0, b0, g0, be0, rm0, rv0,
           W1, b1, g1, be1, rm1, rv1, W2, b2, g2, be2, rm2, rv2, Wf, bf):
    xT = jnp.transpose(coordinate, (0, 2, 1))
    x_pad = jnp.pad(xT, ((0, 0), (0, 5), (0, 0)))
    w0p = jnp.pad(W0, ((0, 0), (0, 5)))
    col = lambda p: p.reshape(-1, 1)
    idx = _run_mlp_topk(
        x_pad, w0p, col(b0), col(g0), col(be0), col(rm0), col(rv0),
        W1, col(b1), col(g1), col(be1), col(rm1), col(rv1),
        W2, col(b2), col(g2), col(be2), col(rm2), col(rv2),
        Wf, col(bf))
    table = jnp.pad(jnp.concatenate([coordinate, feature], axis=-1),
                    ((0, 0), (0, 0), (0, DP - 3 - D))).reshape(B * M, DP)
    rows = _make_gather()(table, idx.reshape(-1)).reshape(B, S, K, DP)
    grouped_points = rows[..., 0:3]
    grouped_feature = rows[..., 3:3 + D]
    return (grouped_points[:, :, 0, :], grouped_points,
            grouped_feature[:, :, 0, :], grouped_feature)


# speedup vs baseline: 11.3560x; 1.0953x over previous
import functools

import jax
import jax.numpy as jnp
from jax import lax
from jax.experimental import pallas as pl
from jax.experimental.pallas import tpu as pltpu
from jax.experimental.pallas import tpu_sc as plsc

B, M, D = 4, 8192, 64
S = 512
K = 32
DP = 128
S_CHUNK = 256

NW = 32
ROWS_PER_W = (S * K) // NW
CHUNK = 256


def _mlp_topk_kernel(x_ref, w0_ref, b0_ref, g0_ref, be0_ref, rm0_ref, rv0_ref,
                     w1_ref, b1_ref, g1_ref, be1_ref, rm1_ref, rv1_ref,
                     w2_ref, b2_ref, g2_ref, be2_ref, rm2_ref, rv2_ref,
                     wf_ref, bf_ref, idx_ref):
    h = x_ref[...]
    layers = ((w0_ref, b0_ref, g0_ref, be0_ref, rm0_ref, rv0_ref),
              (w1_ref, b1_ref, g1_ref, be1_ref, rm1_ref, rv1_ref),
              (w2_ref, b2_ref, g2_ref, be2_ref, rm2_ref, rv2_ref))
    for (w, bb, g, be, rm, rv) in layers:
        y = lax.dot_general(w[...], h, (((1,), (0,)), ((), ())),
                            preferred_element_type=jnp.float32)
        y = y + bb[...]
        y = (y - rm[...]) / jnp.sqrt(rv[...] + 1e-5) * g[...] + be[...]
        h = jnp.maximum(y, 0.0)
    iota = lax.broadcasted_iota(jnp.int32, (S_CHUNK, M), 1)
    nchunks = S // S_CHUNK

    def _score(c):
        s0 = c * S_CHUNK
        q = lax.dot_general(wf_ref[s0:s0 + S_CHUNK, :], h,
                            (((1,), (0,)), ((), ())),
                            preferred_element_type=jnp.float32)
        return jax.nn.sigmoid(q + bf_ref[s0:s0 + S_CHUNK, :])

    q = _score(0)
    for c in range(nchunks):
        q_next = _score(c + 1) if c + 1 < nchunks else None
        cols = []
        for _ in range(K):
            mx = jnp.max(q, axis=1, keepdims=True)
            cand = jnp.where(q == mx, iota, M)
            sel = jnp.min(cand, axis=1, keepdims=True)
            cols.append(sel)
            q = jnp.where(iota == sel, -1.0, q)
        s0 = c * S_CHUNK
        idx_ref[s0:s0 + S_CHUNK, :] = jnp.concatenate(cols, axis=1)
        q = q_next


def _run_mlp_topk(x_pad, w0p, b0, g0, be0, rm0, rv0, w1, b1, g1, be1, rm1, rv1,
                  w2, b2, g2, be2, rm2, rv2, wf, bf):
    return pl.pallas_call(
        _mlp_topk_kernel,
        out_shape=jax.ShapeDtypeStruct((S, K), jnp.int32),
    )(x_pad, w0p, b0, g0, be0, rm0, rv0, w1, b1, g1, be1, rm1, rv1,
      w2, b2, g2, be2, rm2, rv2, wf, bf)


@functools.cache
def _make_gather():
    mesh = plsc.VectorSubcoreMesh(core_axis_name="c", subcore_axis_name="s")

    @functools.partial(
        pl.kernel, mesh=mesh,
        out_type=jax.ShapeDtypeStruct((S * K, DP), jnp.float32),
        scratch_types=[
            pltpu.VMEM((CHUNK,), jnp.int32),
            pltpu.VMEM((CHUNK,), jnp.int32),
            pltpu.VMEM((CHUNK, DP), jnp.float32),
            pltpu.VMEM((CHUNK, DP), jnp.float32),
            pltpu.SemaphoreType.DMA,
            pltpu.SemaphoreType.DMA,
        ],
    )
    def gather(table_hbm, idx_hbm, out_hbm,
               idx_v0, idx_v1, rows_v0, rows_v1, sem0, sem1):
        wid = lax.axis_index("s") * 2 + lax.axis_index("c")
        base = wid * ROWS_PER_W
        nstep = ROWS_PER_W // CHUNK
        idx_b = (idx_v0, idx_v1)
        rows_b = (rows_v0, rows_v1)
        sem_b = (sem0, sem1)
        pltpu.sync_copy(idx_hbm.at[pl.ds(base, CHUNK)], idx_v0)
        cps = [pltpu.async_copy(table_hbm.at[idx_v0], rows_v0, sem0)]
        for cstep in range(nstep):
            if cstep + 1 < nstep:
                noff = base + (cstep + 1) * CHUNK
                nb = (cstep + 1) % 2
                pltpu.sync_copy(idx_hbm.at[pl.ds(noff, CHUNK)], idx_b[nb])
                cps.append(pltpu.async_copy(
                    table_hbm.at[idx_b[nb]], rows_b[nb], sem_b[nb]))
            cps[cstep].wait()
            pltpu.sync_copy(rows_b[cstep % 2],
                            out_hbm.at[pl.ds(base + cstep * CHUNK, CHUNK)])

    return gather


def kernel(coordinate, feature, W0, b0, g0, be0, rm0, rv0,
           W1, b1, g1, be1, rm1, rv1, W2, b2, g2, be2, rm2, rv2, Wf, bf):
    xT = jnp.transpose(coordinate, (0, 2, 1))
    x_pad = jnp.pad(xT, ((0, 0), (0, 5), (0, 0)))
    w0p = jnp.pad(W0, ((0, 0), (0, 5)))
    col = lambda p: p.reshape(-1, 1)
    table = jnp.pad(jnp.concatenate([coordinate, feature], axis=-1),
                    ((0, 0), (0, 0), (0, DP - 3 - D))).reshape(B * M, DP)
    gather = _make_gather()
    parts = []
    for b in range(B):
        idx_b = _run_mlp_topk(
            x_pad[b], w0p, col(b0), col(g0), col(be0), col(rm0), col(rv0),
            W1, col(b1), col(g1), col(be1), col(rm1), col(rv1),
            W2, col(b2), col(g2), col(be2), col(rm2), col(rv2),
            Wf, col(bf))
        parts.append(gather(table, idx_b.reshape(-1) + b * M))
    rows = jnp.stack(parts).reshape(B, S, K, DP)
    grouped_points = rows[..., 0:3]
    grouped_feature = rows[..., 3:3 + D]
    return (grouped_points[:, :, 0, :], grouped_points,
            grouped_feature[:, :, 0, :], grouped_feature)
